# baseline (speedup 1.0000x reference)
import functools

import jax
import jax.numpy as jnp
from jax import lax
from jax.experimental import pallas as pl
from jax.experimental.pallas import tpu as pltpu
from jax.experimental.pallas import tpu_sc as plsc

D = 128
HD = 64
TW = D + HD
G = 16
L = 16
NC = 2
NS = 16
NW = NC * NS
B = 32
NG = B // L
IW = 3 * B
NEG_INF = float("-inf")


def _table_body(x_ref, w_ref, b_ref, o_ref):
    h = (
        jnp.dot(x_ref[...], w_ref[...].T, preferred_element_type=jnp.float32)
        + b_ref[...]
    )
    eh = jnp.exp(h).astype(jnp.bfloat16)
    lo = lax.bitcast_convert_type(eh[:, :HD], jnp.uint16).astype(jnp.int32)
    hi = lax.bitcast_convert_type(eh[:, HD:], jnp.uint16).astype(jnp.int32)
    packed = lax.bitcast_convert_type(lo | (hi << 16), jnp.float32)
    o_ref[...] = jnp.concatenate([h, packed], axis=1)


def _make_table(x, W, b):
    N = x.shape[0]
    return pl.pallas_call(
        _table_body,
        out_shape=jax.ShapeDtypeStruct((N, TW), jnp.float32),
    )(x, W, b[None, :])


def _unpack_eh(w):
    b32 = plsc.bitcast(w, jnp.bfloat16)
    return plsc.unpack(b32, format=plsc.PackFormat.INTERLEAVED)


def _make_edge_kernel(NBW, NPAD):
    mesh = plsc.VectorSubcoreMesh(core_axis_name="c", subcore_axis_name="s")

    @functools.partial(
        pl.kernel,
        mesh=mesh,
        compiler_params=pltpu.CompilerParams(
            needs_layout_passes=False, use_tc_tiling_on_sc=False),
        out_type=[
            jax.ShapeDtypeStruct((NW, G * D), jnp.float32),
            jax.ShapeDtypeStruct((NW, G * D), jnp.float32),
        ],
        scratch_types=[
            pltpu.VMEM((NPAD,), jnp.int32),
            pltpu.VMEM((IW,), jnp.int32),
            pltpu.VMEM((IW,), jnp.int32),
            pltpu.VMEM((2 * B, TW), jnp.float32),
            pltpu.VMEM((2 * B, TW), jnp.float32),
            pltpu.VMEM((D * B,), jnp.float32),
            pltpu.VMEM((L * G * D,), jnp.float32),
            pltpu.VMEM((L * G * D,), jnp.float32),
            pltpu.VMEM((G * D,), jnp.float32),
            pltpu.VMEM((G * D,), jnp.float32),
            pltpu.VMEM((272,), jnp.float32),
            pltpu.SemaphoreType.DMA,
            pltpu.SemaphoreType.DMA,
            pltpu.SemaphoreType.DMA,
            pltpu.SemaphoreType.DMA,
        ],
    )
    def edge_kernel(hh_hbm, idxc_hbm, batch_hbm, params_hbm,
                    out_s_hbm, out_m_hbm,
                    batch_v, idx0_v, idx1_v, rows0, rows1, ar_s,
                    accs, accm, stag_s, stag_m, params_v,
                    semr0, semr1, semi0, semi1):
        wid = lax.axis_index("s") * NC + lax.axis_index("c")
        pltpu.sync_copy(batch_hbm, batch_v)
        pltpu.sync_copy(params_hbm, params_v)

        iota = lax.iota(jnp.int32, L)
        zeros16 = jnp.zeros((L,), jnp.float32)
        neginf16 = jnp.full((L,), NEG_INF, jnp.float32)
        idx_bufs = (idx0_v, idx1_v)
        rows_bufs = (rows0, rows1)
        semr = (semr0, semr1)
        semi = (semi0, semi1)

        def init_body(i, _):
            off = i * (8 * L)
            for j in range(8):
                accs[pl.ds(off + j * L, L)] = zeros16
                accm[pl.ds(off + j * L, L)] = neginf16
            return 0
        lax.fori_loop(0, (L * G * D) // (8 * L), init_body, 0)

        row_s = [iota + g * L for g in range(NG)]
        row_d = [iota + B + g * L for g in range(NG)]
        lane_base = iota * (G * D)
        br_vec = plsc.load_gather(params_v, [jnp.full((L,), 256, jnp.int32)])
        base0 = wid * NBW

        pltpu.sync_copy(idxc_hbm.at[pl.ds(base0 * IW, IW)], idx0_v)
        pltpu.async_copy(
            hh_hbm.at[idx0_v.at[pl.ds(0, 2 * B)]], rows0, semr0)
        pltpu.async_copy(
            idxc_hbm.at[pl.ds((base0 + 1) * IW, IW)], idx1_v, semi1)

        def process(blk, p):
            cur_i, nxt_i = idx_bufs[p], idx_bufs[1 - p]
            cur_r, nxt_r = rows_bufs[p], rows_bufs[1 - p]
            pltpu.make_async_copy(
                idxc_hbm.at[pl.ds(0, IW)], nxt_i, semi[1 - p]).wait()
            pltpu.async_copy(
                hh_hbm.at[nxt_i.at[pl.ds(0, 2 * B)]], nxt_r, semr[1 - p])
            segs, valids, masks = [], [], []
            for g in range(NG):
                src16 = cur_i[pl.ds(g * L, L)]
                segs.append(plsc.load_gather(batch_v, [src16]))
                vi = cur_i[pl.ds(2 * B + g * L, L)]
                masks.append(vi > 0)
                valids.append(vi.astype(jnp.float32))
            pltpu.async_copy(
                idxc_hbm.at[pl.ds((base0 + blk + 2) * IW, IW)],
                cur_i, semi[p])
            pltpu.make_async_copy(
                hh_hbm.at[pl.ds(0, 2 * B)], cur_r, semr[p]).wait()

            def p1_body(j, raccs):
                spl = jnp.full((L,), 0, jnp.int32) + j
                attn0 = plsc.load_gather(params_v, [spl])
                attn1 = plsc.load_gather(params_v, [spl + HD])
                wr0 = plsc.load_gather(params_v, [spl + D])
                wr1 = plsc.load_gather(params_v, [spl + D + HD])
                out = []
                for g in range(NG):
                    cs0 = plsc.load_gather(cur_r, [row_s[g], spl])
                    cd0 = plsc.load_gather(cur_r, [row_d[g], spl])
                    cs1 = plsc.load_gather(cur_r, [row_s[g], spl + HD])
                    cd1 = plsc.load_gather(cur_r, [row_d[g], spl + HD])
                    ws = plsc.load_gather(cur_r, [row_s[g], spl + D])
                    wd = plsc.load_gather(cur_r, [row_d[g], spl + D])
                    es_lo, es_hi = _unpack_eh(ws)
                    ed_lo, ed_hi = _unpack_eh(wd)
                    s0 = cs0 + cd0
                    s1 = cs1 + cd1
                    e0 = es_lo * ed_lo
                    e1 = es_hi * ed_hi
                    elu0 = jnp.where(s0 > 0, s0, e0 - 1.0)
                    elu1 = jnp.where(s1 > 0, s1, e1 - 1.0)
                    t0 = elu0 * attn0
                    t1 = elu1 * attn1
                    ar0 = jnp.where(t0 > 0, t0, jnp.exp(t0) - 1.0)
                    ar1 = jnp.where(t1 > 0, t1, jnp.exp(t1) - 1.0)
                    ar_s[pl.ds(j * B + g * L, L)] = ar0
                    ar_s[pl.ds((j + HD) * B + g * L, L)] = ar1
                    out.append(raccs[g] + ar0 * wr0 + ar1 * wr1)
                return tuple(out)

            raccs = lax.fori_loop(
                0, HD, p1_body, tuple(zeros16 for _ in range(NG)))

            scores, bases = [], []
            for g in range(NG):
                w = raccs[g] + br_vec
                score = 1.0 / (1.0 + jnp.exp(-w))
                scores.append(score * valids[g])
                bases.append(lane_base + segs[g] * D)

            def p2_body(j, _):
                spl = jnp.full((L,), 0, jnp.int32) + j
                for g in range(NG):
                    ar0 = ar_s[pl.ds(j * B + g * L, L)]
                    ar1 = ar_s[pl.ds((j + HD) * B + g * L, L)]
                    idx0 = bases[g] + spl
                    idx1 = idx0 + HD
                    plsc.addupdate_scatter(accs, [idx0], ar0 * scores[g])
                    plsc.addupdate_scatter(accs, [idx1], ar1 * scores[g])
                    arm0 = jnp.where(masks[g], ar0, NEG_INF)
                    arm1 = jnp.where(masks[g], ar1, NEG_INF)
                    old0 = plsc.load_gather(accm, [idx0])
                    old1 = plsc.load_gather(accm, [idx1])
                    plsc.store_scatter(accm, [idx0], jnp.maximum(old0, arm0))
                    plsc.store_scatter(accm, [idx1], jnp.maximum(old1, arm1))
                return 0

            lax.fori_loop(0, HD, p2_body, 0)

        def block_body(i, _):
            for p in range(2):
                process(2 * i + p, p)
            return 0
        lax.fori_loop(0, NBW // 2, block_body, 0)

        pltpu.make_async_copy(hh_hbm.at[pl.ds(0, 2 * B)], rows0, semr0).wait()
        pltpu.make_async_copy(idxc_hbm.at[pl.ds(0, IW)], idx1_v, semi1).wait()

        def red_body(i, _):
            sd = (i >> 3) * D + (i & 7) * L

            def lred(l, carry):
                a, m = carry
                off = l * (G * D) + sd
                return (a + accs[pl.ds(off, L)],
                        jnp.maximum(m, accm[pl.ds(off, L)]))

            a, m = lax.fori_loop(0, L, lred, (zeros16, neginf16))
            stag_s[pl.ds(sd, L)] = a
            stag_m[pl.ds(sd, L)] = m
            return 0

        lax.fori_loop(0, G * (D // L), red_body, 0)

        pltpu.sync_copy(stag_s, out_s_hbm.at[wid])
        pltpu.sync_copy(stag_m, out_m_hbm.at[wid])

    return edge_kernel


def kernel(x, edge_index, edge_attr, pos, batch, W, b, attn, Wr, br):
    num_nodes = x.shape[0]
    E = edge_index.shape[1]

    a = jnp.minimum(edge_index[0], edge_index[1])
    bb = jnp.maximum(edge_index[0], edge_index[1])
    ids = a * num_nodes + bb
    ids_sorted = jnp.sort(ids)
    keep = jnp.concatenate(
        [jnp.ones((1,), dtype=bool), ids_sorted[1:] != ids_sorted[:-1]])
    a_s = (ids_sorted // num_nodes).astype(jnp.int32)
    b_s = (ids_sorted % num_nodes).astype(jnp.int32)
    present = jax.ops.segment_sum(
        jnp.ones((2 * E,), jnp.float32), edge_index.ravel(),
        num_segments=num_nodes) > 0.5

    all_nodes = jnp.arange(num_nodes, dtype=jnp.int32)
    ET = E + num_nodes
    NBW = -(-ET // (NW * B))
    NBW += NBW % 2
    PW = NBW * B
    EP = NW * PW
    pad = EP - ET
    src_pad = jnp.concatenate([a_s, all_nodes, jnp.zeros((pad,), jnp.int32)])
    dst_pad = jnp.concatenate([b_s, all_nodes, jnp.zeros((pad,), jnp.int32)])
    val_pad = jnp.concatenate(
        [keep, ~present, jnp.zeros((pad,), bool)]).astype(jnp.int32)
    NBT = NW * NBW
    idxc = jnp.concatenate(
        [src_pad.reshape(NBT, B), dst_pad.reshape(NBT, B),
         val_pad.reshape(NBT, B)], axis=1)
    idxc = jnp.concatenate(
        [idxc.reshape(-1), jnp.zeros((2 * IW,), jnp.int32)])

    NPAD = -(-num_nodes // 64) * 64
    batch_pad = jnp.concatenate(
        [batch.astype(jnp.int32),
         jnp.zeros((NPAD - num_nodes,), jnp.int32)])
    params = jnp.concatenate(
        [attn[0].astype(jnp.float32), Wr[0].astype(jnp.float32),
         br.astype(jnp.float32), jnp.zeros((272 - 2 * D - 1,), jnp.float32)])

    hh = _make_table(x, W, b)
    edge_kernel = _make_edge_kernel(NBW, NPAD)
    out_s, out_m = edge_kernel(hh, idxc, batch_pad, params)

    out1 = out_s.sum(axis=0).reshape(G, D)
    out2 = out_m.max(axis=0).reshape(G, D)
    return jnp.concatenate([out1, out2], axis=1)


# speedup vs baseline: 2.8525x; 2.8525x over previous
import functools

import jax
import jax.numpy as jnp
from jax import lax
from jax.experimental import pallas as pl
from jax.experimental.pallas import tpu as pltpu
from jax.experimental.pallas import tpu_sc as plsc

D = 128
HD = 64
TW = D + HD
NK = D // 16
G = 16
L = 16
NC = 2
NW = NC * 16
B = 64
NG = B // L
IW = 3 * B
NEG_INF = float("-inf")


def _table_body(x_ref, w_ref, b_ref, o_ref):
    h = (
        jnp.dot(x_ref[...], w_ref[...].T, preferred_element_type=jnp.float32)
        + b_ref[...]
    )
    eh = jnp.exp(h).astype(jnp.bfloat16)
    lo = lax.bitcast_convert_type(eh[:, :HD], jnp.uint16).astype(jnp.int32)
    hi = lax.bitcast_convert_type(eh[:, HD:], jnp.uint16).astype(jnp.int32)
    packed = lax.bitcast_convert_type(lo | (hi << 16), jnp.float32)
    o_ref[...] = jnp.concatenate([h, packed], axis=1)


def _make_table(x, W, b):
    N = x.shape[0]
    return pl.pallas_call(
        _table_body,
        out_shape=jax.ShapeDtypeStruct((N, TW), jnp.float32),
    )(x, W, b[None, :])


def _splat(v, lane):
    idx = jnp.full((L,), 0, jnp.int32) + lane
    return v.at[idx].get(mode="promise_in_bounds")


def _make_edge_kernel(NBW, NPAD):
    mesh = plsc.VectorSubcoreMesh(core_axis_name="c", subcore_axis_name="s")

    @functools.partial(
        pl.kernel,
        mesh=mesh,
        compiler_params=pltpu.CompilerParams(
            needs_layout_passes=False, use_tc_tiling_on_sc=False),
        out_type=[
            jax.ShapeDtypeStruct((NW, G * D), jnp.float32),
            jax.ShapeDtypeStruct((NW, G * D), jnp.float32),
        ],
        scratch_types=[
            pltpu.VMEM((NPAD,), jnp.int32),
            pltpu.VMEM((IW,), jnp.int32),
            pltpu.VMEM((IW,), jnp.int32),
            pltpu.VMEM((2 * B, TW), jnp.float32),
            pltpu.VMEM((2 * B, TW), jnp.float32),
            pltpu.VMEM((G * D,), jnp.float32),
            pltpu.VMEM((G * D,), jnp.float32),
            pltpu.VMEM((272,), jnp.float32),
            pltpu.SemaphoreType.DMA,
            pltpu.SemaphoreType.DMA,
            pltpu.SemaphoreType.DMA,
            pltpu.SemaphoreType.DMA,
        ],
    )
    def edge_kernel(hh_hbm, idxc_hbm, batch_hbm, params_hbm,
                    out_s_hbm, out_m_hbm,
                    batch_v, idx0_v, idx1_v, rows0, rows1,
                    accs, accm, params_v,
                    semr0, semr1, semi0, semi1):
        wid = lax.axis_index("s") * NC + lax.axis_index("c")
        pltpu.sync_copy(batch_hbm, batch_v)
        pltpu.sync_copy(params_hbm, params_v)

        iota = lax.iota(jnp.int32, L)
        zeros16 = jnp.zeros((L,), jnp.float32)
        neginf16 = jnp.full((L,), NEG_INF, jnp.float32)
        idx_bufs = (idx0_v, idx1_v)
        rows_bufs = (rows0, rows1)
        semr = (semr0, semr1)
        semi = (semi0, semi1)

        def init_body(i, _):
            off = i * (8 * L)
            for j in range(8):
                accs[pl.ds(off + j * L, L)] = zeros16
                accm[pl.ds(off + j * L, L)] = neginf16
            return 0
        lax.fori_loop(0, (G * D) // (8 * L), init_body, 0)

        attn_v = [params_v[pl.ds(k * L, L)] for k in range(NK)]
        wr_v = [params_v[pl.ds(D + k * L, L)] for k in range(NK)]
        br_vec = plsc.load_gather(params_v, [jnp.full((L,), 256, jnp.int32)])
        chunk_iota = [iota + k * L for k in range(NK)]
        base0 = wid * NBW

        pltpu.sync_copy(idxc_hbm.at[pl.ds(base0 * IW, IW)], idx0_v)
        pltpu.async_copy(
            hh_hbm.at[idx0_v.at[pl.ds(0, 2 * B)]], rows0, semr0)
        pltpu.async_copy(
            idxc_hbm.at[pl.ds((base0 + 1) * IW, IW)], idx1_v, semi1)

        def edge_work(cur_r, g, ep, segs, validfs):
            row = g * L + ep
            hs = [cur_r[row, pl.ds(k * L, L)] for k in range(NK)]
            hd = [cur_r[B + row, pl.ds(k * L, L)] for k in range(NK)]
            ws = [cur_r[row, pl.ds(D + j * L, L)] for j in range(NK // 2)]
            wd = [cur_r[B + row, pl.ds(D + j * L, L)] for j in range(NK // 2)]
            es = [None] * NK
            ed = [None] * NK
            for j in range(NK // 2):
                lo, hi = plsc.unpack(
                    plsc.bitcast(ws[j], jnp.bfloat16),
                    format=plsc.PackFormat.INTERLEAVED)
                es[j], es[j + NK // 2] = lo, hi
                lo, hi = plsc.unpack(
                    plsc.bitcast(wd[j], jnp.bfloat16),
                    format=plsc.PackFormat.INTERLEAVED)
                ed[j], ed[j + NK // 2] = lo, hi
            ar = []
            dots = []
            for k in range(NK):
                s = hs[k] + hd[k]
                e = es[k] * ed[k]
                elu1 = jnp.where(s > 0, s, e - 1.0)
                t = elu1 * attn_v[k]
                a = jnp.where(t > 0, t, jnp.exp(t) - 1.0)
                ar.append(a)
                dots.append(a * wr_v[k])
            tot = jnp.sum(
                ((dots[0] + dots[1]) + (dots[2] + dots[3]))
                + ((dots[4] + dots[5]) + (dots[6] + dots[7])))
            valsp = _splat(validfs[g], ep)
            segsp = _splat(segs[g], ep)
            w = br_vec + tot
            score = (1.0 / (1.0 + jnp.exp(-w))) * valsp
            mask = valsp > 0
            base = segsp * D
            for k in range(NK):
                idx = base + chunk_iota[k]
                plsc.addupdate_scatter(accs, [idx], ar[k] * score)
                arm = jnp.where(mask, ar[k], NEG_INF)
                old = plsc.load_gather(accm, [idx])
                plsc.store_scatter(accm, [idx], jnp.maximum(old, arm))

        def process(blk, p):
            cur_i, nxt_i = idx_bufs[p], idx_bufs[1 - p]
            cur_r, nxt_r = rows_bufs[p], rows_bufs[1 - p]
            pltpu.make_async_copy(
                idxc_hbm.at[pl.ds(0, IW)], nxt_i, semi[1 - p]).wait()
            pltpu.async_copy(
                hh_hbm.at[nxt_i.at[pl.ds(0, 2 * B)]], nxt_r, semr[1 - p])
            segs, validfs = [], []
            for g in range(NG):
                src16 = cur_i[pl.ds(g * L, L)]
                segs.append(plsc.load_gather(batch_v, [src16]))
                vi = cur_i[pl.ds(2 * B + g * L, L)]
                validfs.append(vi.astype(jnp.float32))
            pltpu.async_copy(
                idxc_hbm.at[pl.ds((base0 + blk + 2) * IW, IW)],
                cur_i, semi[p])
            pltpu.make_async_copy(
                hh_hbm.at[pl.ds(0, 2 * B)], cur_r, semr[p]).wait()

            def lane_body(ep, _):
                for gp in range(2):
                    edge_work(cur_r, gp, ep, segs, validfs)
                    edge_work(cur_r, gp + 2, ep, segs, validfs)
                return 0

            lax.fori_loop(0, L, lane_body, 0)

        def block_body(i, _):
            for p in range(2):
                process(2 * i + p, p)
            return 0
        lax.fori_loop(0, NBW // 2, block_body, 0)

        pltpu.make_async_copy(hh_hbm.at[pl.ds(0, 2 * B)], rows0, semr0).wait()
        pltpu.make_async_copy(idxc_hbm.at[pl.ds(0, IW)], idx1_v, semi1).wait()

        pltpu.sync_copy(accs, out_s_hbm.at[wid])
        pltpu.sync_copy(accm, out_m_hbm.at[wid])

    return edge_kernel


def kernel(x, edge_index, edge_attr, pos, batch, W, b, attn, Wr, br):
    num_nodes = x.shape[0]
    E = edge_index.shape[1]

    a = jnp.minimum(edge_index[0], edge_index[1])
    bb = jnp.maximum(edge_index[0], edge_index[1])
    ids = a * num_nodes + bb
    ids_sorted = jnp.sort(ids)
    keep = jnp.concatenate(
        [jnp.ones((1,), dtype=bool), ids_sorted[1:] != ids_sorted[:-1]])
    a_s = (ids_sorted // num_nodes).astype(jnp.int32)
    b_s = (ids_sorted % num_nodes).astype(jnp.int32)
    present = jax.ops.segment_sum(
        jnp.ones((2 * E,), jnp.float32), edge_index.ravel(),
        num_segments=num_nodes) > 0.5

    all_nodes = jnp.arange(num_nodes, dtype=jnp.int32)
    ET = E + num_nodes
    NBW = -(-ET // (NW * B))
    NBW += NBW % 2
    PW = NBW * B
    EP = NW * PW
    pad = EP - ET
    src_pad = jnp.concatenate([a_s, all_nodes, jnp.zeros((pad,), jnp.int32)])
    dst_pad = jnp.concatenate([b_s, all_nodes, jnp.zeros((pad,), jnp.int32)])
    val_pad = jnp.concatenate(
        [keep, ~present, jnp.zeros((pad,), bool)]).astype(jnp.int32)
    NBT = NW * NBW
    idxc = jnp.concatenate(
        [src_pad.reshape(NBT, B), dst_pad.reshape(NBT, B),
         val_pad.reshape(NBT, B)], axis=1)
    idxc = jnp.concatenate(
        [idxc.reshape(-1), jnp.zeros((2 * IW,), jnp.int32)])

    NPAD = -(-num_nodes // 64) * 64
    batch_pad = jnp.concatenate(
        [batch.astype(jnp.int32),
         jnp.zeros((NPAD - num_nodes,), jnp.int32)])
    params = jnp.concatenate(
        [attn[0].astype(jnp.float32), Wr[0].astype(jnp.float32),
         br.astype(jnp.float32), jnp.zeros((272 - 2 * D - 1,), jnp.float32)])

    hh = _make_table(x, W, b)
    edge_kernel = _make_edge_kernel(NBW, NPAD)
    out_s, out_m = edge_kernel(hh, idxc, batch_pad, params)

    out1 = out_s.sum(axis=0).reshape(G, D)
    out2 = out_m.max(axis=0).reshape(G, D)
    return jnp.concatenate([out1, out2], axis=1)


# speedup vs baseline: 3.2919x; 1.1541x over previous
import functools

import jax
import jax.numpy as jnp
from jax import lax
from jax.experimental import pallas as pl
from jax.experimental.pallas import tpu as pltpu
from jax.experimental.pallas import tpu_sc as plsc

D = 128
HD = 64
TW = D + HD
NK = D // 16
G = 16
L = 16
NC = 2
NW = NC * 16
B = 64
NG = B // L
IW = 3 * B
NEG_INF = float("-inf")


def _table_body(x_ref, w_ref, b_ref, o_ref):
    h = (
        jnp.dot(x_ref[...], w_ref[...].T, preferred_element_type=jnp.float32)
        + b_ref[...]
    )
    eh = jnp.exp(h).astype(jnp.bfloat16)
    lo = lax.bitcast_convert_type(eh[:, :HD], jnp.uint16).astype(jnp.int32)
    hi = lax.bitcast_convert_type(eh[:, HD:], jnp.uint16).astype(jnp.int32)
    packed = lax.bitcast_convert_type(lo | (hi << 16), jnp.float32)
    o_ref[...] = jnp.concatenate([h, packed], axis=1)


def _make_table(x, W, b):
    N = x.shape[0]
    return pl.pallas_call(
        _table_body,
        out_shape=jax.ShapeDtypeStruct((N, TW), jnp.float32),
    )(x, W, b[None, :])


def _splat(v, lane):
    idx = jnp.full((L,), 0, jnp.int32) + lane
    return v.at[idx].get(mode="promise_in_bounds")


def _make_edge_kernel(NBW, NPAD):
    mesh = plsc.VectorSubcoreMesh(core_axis_name="c", subcore_axis_name="s")

    @functools.partial(
        pl.kernel,
        mesh=mesh,
        compiler_params=pltpu.CompilerParams(
            needs_layout_passes=False, use_tc_tiling_on_sc=False),
        out_type=[
            jax.ShapeDtypeStruct((NW, G * D), jnp.float32),
            jax.ShapeDtypeStruct((NW, G * D), jnp.float32),
        ],
        scratch_types=[
            pltpu.VMEM((NPAD,), jnp.int32),
            pltpu.VMEM((IW,), jnp.int32),
            pltpu.VMEM((IW,), jnp.int32),
            pltpu.VMEM((2 * B, TW), jnp.float32),
            pltpu.VMEM((2 * B, TW), jnp.float32),
            pltpu.VMEM((G * D,), jnp.float32),
            pltpu.VMEM((G * D,), jnp.float32),
            pltpu.VMEM((272,), jnp.float32),
            pltpu.SemaphoreType.DMA,
            pltpu.SemaphoreType.DMA,
            pltpu.SemaphoreType.DMA,
            pltpu.SemaphoreType.DMA,
        ],
    )
    def edge_kernel(hh_hbm, idxc_hbm, batch_hbm, params_hbm,
                    out_s_hbm, out_m_hbm,
                    batch_v, idx0_v, idx1_v, rows0, rows1,
                    accs, accm, params_v,
                    semr0, semr1, semi0, semi1):
        wid = lax.axis_index("s") * NC + lax.axis_index("c")
        pltpu.sync_copy(batch_hbm, batch_v)
        pltpu.sync_copy(params_hbm, params_v)

        iota = lax.iota(jnp.int32, L)
        zeros16 = jnp.zeros((L,), jnp.float32)
        neginf16 = jnp.full((L,), NEG_INF, jnp.float32)
        idx_bufs = (idx0_v, idx1_v)
        rows_bufs = (rows0, rows1)
        semr = (semr0, semr1)
        semi = (semi0, semi1)

        def init_body(i, _):
            off = i * (8 * L)
            for j in range(8):
                accs[pl.ds(off + j * L, L)] = zeros16
                accm[pl.ds(off + j * L, L)] = neginf16
            return 0
        lax.fori_loop(0, (G * D) // (8 * L), init_body, 0)

        attn_v = [params_v[pl.ds(k * L, L)] for k in range(NK)]
        wr_v = [params_v[pl.ds(D + k * L, L)] for k in range(NK)]
        br_vec = plsc.load_gather(params_v, [jnp.full((L,), 256, jnp.int32)])
        chunk_iota = [iota + k * L for k in range(NK)]
        base0 = wid * NBW

        pltpu.sync_copy(idxc_hbm.at[pl.ds(base0 * IW, IW)], idx0_v)
        pltpu.async_copy(
            hh_hbm.at[idx0_v.at[pl.ds(0, 2 * B)]], rows0, semr0)
        pltpu.async_copy(
            idxc_hbm.at[pl.ds((base0 + 1) * IW, IW)], idx1_v, semi1)

        def flush(prev, aS, aM):
            base = prev * D
            for k in range(NK):
                idx = base + chunk_iota[k]
                plsc.addupdate_scatter(accs, [idx], aS[k])
                old = plsc.load_gather(accm, [idx])
                plsc.store_scatter(accm, [idx], jnp.maximum(old, aM[k]))

        def edge_work(cur_r, g, ep, segs, validfs, carry):
            prev, aS, aM = carry[0], carry[1:1 + NK], carry[1 + NK:]
            row = g * L + ep
            hs = [cur_r[row, pl.ds(k * L, L)] for k in range(NK)]
            hd = [cur_r[B + row, pl.ds(k * L, L)] for k in range(NK)]
            ws = [cur_r[row, pl.ds(D + j * L, L)] for j in range(NK // 2)]
            wd = [cur_r[B + row, pl.ds(D + j * L, L)] for j in range(NK // 2)]
            es = [None] * NK
            ed = [None] * NK
            for j in range(NK // 2):
                lo, hi = plsc.unpack(
                    plsc.bitcast(ws[j], jnp.bfloat16),
                    format=plsc.PackFormat.INTERLEAVED)
                es[j], es[j + NK // 2] = lo, hi
                lo, hi = plsc.unpack(
                    plsc.bitcast(wd[j], jnp.bfloat16),
                    format=plsc.PackFormat.INTERLEAVED)
                ed[j], ed[j + NK // 2] = lo, hi
            ar = []
            dots = []
            for k in range(NK):
                s = hs[k] + hd[k]
                e = es[k] * ed[k]
                elu1 = jnp.where(s > 0, s, e - 1.0)
                t = elu1 * attn_v[k]
                a = jnp.where(t > 0, t, jnp.exp(t) - 1.0)
                ar.append(a)
                dots.append(a * wr_v[k])
            tot = jnp.sum(
                ((dots[0] + dots[1]) + (dots[2] + dots[3]))
                + ((dots[4] + dots[5]) + (dots[6] + dots[7])))
            valsp = _splat(validfs[g], ep)
            segsp = _splat(segs[g], ep)
            w = br_vec + tot
            score = (1.0 / (1.0 + jnp.exp(-w))) * valsp
            mask = valsp > 0

            changed = jnp.any(segsp != prev)

            def do_flush(prev, aS, aM):
                flush(prev, aS, aM)
                return ([jnp.zeros((L,), jnp.float32)] * NK,
                        [jnp.full((L,), NEG_INF, jnp.float32)] * NK)

            aS, aM = lax.cond(
                changed, do_flush, lambda p, s, m: (list(s), list(m)),
                prev, tuple(aS), tuple(aM))
            aS = [aS[k] + ar[k] * score for k in range(NK)]
            aM = [jnp.maximum(aM[k], jnp.where(mask, ar[k], NEG_INF))
                  for k in range(NK)]
            return (segsp, *aS, *aM)

        def process(blk, p, carry):
            cur_i, nxt_i = idx_bufs[p], idx_bufs[1 - p]
            cur_r, nxt_r = rows_bufs[p], rows_bufs[1 - p]
            pltpu.make_async_copy(
                idxc_hbm.at[pl.ds(0, IW)], nxt_i, semi[1 - p]).wait()
            pltpu.async_copy(
                hh_hbm.at[nxt_i.at[pl.ds(0, 2 * B)]], nxt_r, semr[1 - p])
            segs, validfs = [], []
            for g in range(NG):
                src16 = cur_i[pl.ds(g * L, L)]
                segs.append(plsc.load_gather(batch_v, [src16]))
                vi = cur_i[pl.ds(2 * B + g * L, L)]
                validfs.append(vi.astype(jnp.float32))
            pltpu.async_copy(
                idxc_hbm.at[pl.ds((base0 + blk + 2) * IW, IW)],
                cur_i, semi[p])
            pltpu.make_async_copy(
                hh_hbm.at[pl.ds(0, 2 * B)], cur_r, semr[p]).wait()

            def lane_body(ep, c):
                for g in range(NG):
                    c = edge_work(cur_r, g, ep, segs, validfs, c)
                return c

            return lax.fori_loop(0, L, lane_body, carry)

        def block_body(i, carry):
            for p in range(2):
                carry = process(2 * i + p, p, carry)
            return carry

        carry0 = (jnp.zeros((L,), jnp.int32),
                  *([zeros16] * NK), *([neginf16] * NK))
        carry = lax.fori_loop(0, NBW // 2, block_body, carry0)
        flush(carry[0], carry[1:1 + NK], carry[1 + NK:])

        pltpu.make_async_copy(hh_hbm.at[pl.ds(0, 2 * B)], rows0, semr0).wait()
        pltpu.make_async_copy(idxc_hbm.at[pl.ds(0, IW)], idx1_v, semi1).wait()

        pltpu.sync_copy(accs, out_s_hbm.at[wid])
        pltpu.sync_copy(accm, out_m_hbm.at[wid])

    return edge_kernel


def kernel(x, edge_index, edge_attr, pos, batch, W, b, attn, Wr, br):
    num_nodes = x.shape[0]
    E = edge_index.shape[1]

    a = jnp.minimum(edge_index[0], edge_index[1])
    bb = jnp.maximum(edge_index[0], edge_index[1])
    ids = a * num_nodes + bb
    ids_sorted = jnp.sort(ids)
    keep = jnp.concatenate(
        [jnp.ones((1,), dtype=bool), ids_sorted[1:] != ids_sorted[:-1]])
    a_s = (ids_sorted // num_nodes).astype(jnp.int32)
    b_s = (ids_sorted % num_nodes).astype(jnp.int32)
    present = jax.ops.segment_sum(
        jnp.ones((2 * E,), jnp.float32), edge_index.ravel(),
        num_segments=num_nodes) > 0.5

    all_nodes = jnp.arange(num_nodes, dtype=jnp.int32)
    ET = E + num_nodes
    NBW = -(-ET // (NW * B))
    NBW += NBW % 2
    PW = NBW * B
    EP = NW * PW
    pad = EP - ET
    src_pad = jnp.concatenate([a_s, all_nodes, jnp.zeros((pad,), jnp.int32)])
    dst_pad = jnp.concatenate([b_s, all_nodes, jnp.zeros((pad,), jnp.int32)])
    val_pad = jnp.concatenate(
        [keep, ~present, jnp.zeros((pad,), bool)]).astype(jnp.int32)
    NBT = NW * NBW
    idxc = jnp.concatenate(
        [src_pad.reshape(NBT, B), dst_pad.reshape(NBT, B),
         val_pad.reshape(NBT, B)], axis=1)
    idxc = jnp.concatenate(
        [idxc.reshape(-1), jnp.zeros((2 * IW,), jnp.int32)])

    NPAD = -(-num_nodes // 64) * 64
    batch_pad = jnp.concatenate(
        [batch.astype(jnp.int32),
         jnp.zeros((NPAD - num_nodes,), jnp.int32)])
    params = jnp.concatenate(
        [attn[0].astype(jnp.float32), Wr[0].astype(jnp.float32),
         br.astype(jnp.float32), jnp.zeros((272 - 2 * D - 1,), jnp.float32)])

    hh = _make_table(x, W, b)
    edge_kernel = _make_edge_kernel(NBW, NPAD)
    out_s, out_m = edge_kernel(hh, idxc, batch_pad, params)

    out1 = out_s.sum(axis=0).reshape(G, D)
    out2 = out_m.max(axis=0).reshape(G, D)
    return jnp.concatenate([out1, out2], axis=1)


# speedup vs baseline: 4.6211x; 1.4038x over previous
import functools

import jax
import jax.numpy as jnp
from jax import lax
from jax.experimental import pallas as pl
from jax.experimental.pallas import tpu as pltpu
from jax.experimental.pallas import tpu_sc as plsc

D = 128
HD = 64
TW = D + HD
NK = D // 16
G = 16
L = 16
NC = 2
NW = NC * 16
B = 64
NG = B // L
IW = 3 * B
NEG_INF = float("-inf")


def _table_body(x_ref, w_ref, b_ref, o_ref):
    h = (
        jnp.dot(x_ref[...], w_ref[...].T, preferred_element_type=jnp.float32)
        + b_ref[...]
    )
    eh = jnp.exp(h).astype(jnp.bfloat16)
    lo = lax.bitcast_convert_type(eh[:, :HD], jnp.uint16).astype(jnp.int32)
    hi = lax.bitcast_convert_type(eh[:, HD:], jnp.uint16).astype(jnp.int32)
    packed = lax.bitcast_convert_type(lo | (hi << 16), jnp.float32)
    o_ref[...] = jnp.concatenate([h, packed], axis=1)


def _make_table(x, W, b):
    N = x.shape[0]
    return pl.pallas_call(
        _table_body,
        out_shape=jax.ShapeDtypeStruct((N, TW), jnp.float32),
    )(x, W, b[None, :])


def _splat(v, lane):
    idx = jnp.full((L,), 0, jnp.int32) + lane
    return v.at[idx].get(mode="promise_in_bounds")


def _make_edge_kernel(NBW, NPAD, E, N):
    mesh = plsc.VectorSubcoreMesh(core_axis_name="c", subcore_axis_name="s")
    EPT = (2 * E) // 16
    ECH = EPT // 4
    PW = NBW * B

    @functools.partial(
        pl.kernel,
        mesh=mesh,
        compiler_params=pltpu.CompilerParams(
            needs_layout_passes=False, use_tc_tiling_on_sc=False),
        out_type=[
            jax.ShapeDtypeStruct((NW, G * D), jnp.float32),
            jax.ShapeDtypeStruct((NW, G * D), jnp.float32),
        ],
        scratch_types=[
            pltpu.VMEM((NPAD,), jnp.int32),
            pltpu.VMEM((IW,), jnp.int32),
            pltpu.VMEM((IW,), jnp.int32),
            pltpu.VMEM((2 * B, TW), jnp.float32),
            pltpu.VMEM((2 * B, TW), jnp.float32),
            pltpu.VMEM((G * D,), jnp.float32),
            pltpu.VMEM((G * D,), jnp.float32),
            pltpu.VMEM((272,), jnp.float32),
            pltpu.VMEM((NPAD,), jnp.int32),
            pltpu.VMEM((ECH,), jnp.int32),
            pltpu.VMEM((ECH,), jnp.int32),
            pltpu.VMEM_SHARED((NPAD,), jnp.int32),
            pltpu.SemaphoreType.DMA,
            pltpu.SemaphoreType.DMA,
            pltpu.SemaphoreType.DMA,
            pltpu.SemaphoreType.DMA,
        ],
    )
    def edge_kernel(hh_hbm, idxc_hbm, batch_hbm, params_hbm, ep_hbm,
                    out_s_hbm, out_m_hbm,
                    batch_v, idx0_v, idx1_v, rows0, rows1,
                    accs, accm, params_v, pres_v, epidx_v, ones_v,
                    pres_sh,
                    semr0, semr1, semi0, semi1):
        sid = lax.axis_index("s")
        wid = sid * NC + lax.axis_index("c")
        pltpu.sync_copy(batch_hbm, batch_v)
        pltpu.sync_copy(params_hbm, params_v)

        zeros16i = jnp.zeros((L,), jnp.int32)
        ones16i = jnp.ones((L,), jnp.int32)

        def zinit_body(i, _):
            off = i * (8 * L)
            for j in range(8):
                pres_v[pl.ds(off + j * L, L)] = zeros16i
            return 0
        lax.fori_loop(0, NPAD // (8 * L), zinit_body, 0)

        def oinit_body(i, _):
            ones_v[pl.ds(i * L, L)] = ones16i
            return 0
        lax.fori_loop(0, ECH // L, oinit_body, 0)

        @pl.when(sid == 0)
        def _():
            pltpu.sync_copy(pres_v, pres_sh)

        plsc.subcore_barrier()
        for c in range(4):
            pltpu.sync_copy(
                ep_hbm.at[pl.ds(sid * EPT + c * ECH, ECH)], epidx_v)
            pltpu.sync_copy(ones_v, pres_sh.at[epidx_v], add=True)
        plsc.subcore_barrier()
        pltpu.sync_copy(pres_sh, pres_v)

        iota = lax.iota(jnp.int32, L)
        zeros16 = jnp.zeros((L,), jnp.float32)
        neginf16 = jnp.full((L,), NEG_INF, jnp.float32)
        idx_bufs = (idx0_v, idx1_v)
        rows_bufs = (rows0, rows1)
        semr = (semr0, semr1)
        semi = (semi0, semi1)

        def init_body(i, _):
            off = i * (8 * L)
            for j in range(8):
                accs[pl.ds(off + j * L, L)] = zeros16
                accm[pl.ds(off + j * L, L)] = neginf16
            return 0
        lax.fori_loop(0, (G * D) // (8 * L), init_body, 0)

        attn_v = [params_v[pl.ds(k * L, L)] for k in range(NK)]
        wr_v = [params_v[pl.ds(D + k * L, L)] for k in range(NK)]
        br_vec = plsc.load_gather(params_v, [jnp.full((L,), 256, jnp.int32)])
        chunk_iota = [iota + k * L for k in range(NK)]
        base0 = wid * NBW

        pltpu.sync_copy(idxc_hbm.at[pl.ds(base0 * IW, IW)], idx0_v)
        pltpu.async_copy(
            hh_hbm.at[idx0_v.at[pl.ds(0, 2 * B)]], rows0, semr0)
        pltpu.async_copy(
            idxc_hbm.at[pl.ds((base0 + 1) * IW, IW)], idx1_v, semi1)

        def flush(prev, aS, aM):
            base = prev * D
            for k in range(NK):
                idx = base + chunk_iota[k]
                plsc.addupdate_scatter(accs, [idx], aS[k])
                old = plsc.load_gather(accm, [idx])
                plsc.store_scatter(accm, [idx], jnp.maximum(old, aM[k]))

        def edge_work(cur_r, g, ep, segs, validfs, carry):
            prev, aS, aM = carry[0], carry[1:1 + NK], carry[1 + NK:]
            row = g * L + ep
            hs = [cur_r[row, pl.ds(k * L, L)] for k in range(NK)]
            hd = [cur_r[B + row, pl.ds(k * L, L)] for k in range(NK)]
            ws = [cur_r[row, pl.ds(D + j * L, L)] for j in range(NK // 2)]
            wd = [cur_r[B + row, pl.ds(D + j * L, L)] for j in range(NK // 2)]
            es = [None] * NK
            ed = [None] * NK
            for j in range(NK // 2):
                lo, hi = plsc.unpack(
                    plsc.bitcast(ws[j], jnp.bfloat16),
                    format=plsc.PackFormat.INTERLEAVED)
                es[j], es[j + NK // 2] = lo, hi
                lo, hi = plsc.unpack(
                    plsc.bitcast(wd[j], jnp.bfloat16),
                    format=plsc.PackFormat.INTERLEAVED)
                ed[j], ed[j + NK // 2] = lo, hi
            ar = []
            dots = []
            for k in range(NK):
                s = hs[k] + hd[k]
                e = es[k] * ed[k]
                elu1 = jnp.where(s > 0, s, e - 1.0)
                t = elu1 * attn_v[k]
                a = jnp.where(t > 0, t, jnp.exp(t) - 1.0)
                ar.append(a)
                dots.append(a * wr_v[k])
            tot = jnp.sum(
                ((dots[0] + dots[1]) + (dots[2] + dots[3]))
                + ((dots[4] + dots[5]) + (dots[6] + dots[7])))
            valsp = _splat(validfs[g], ep)
            segsp = _splat(segs[g], ep)
            w = br_vec + tot
            score = (1.0 / (1.0 + jnp.exp(-w))) * valsp
            mask = valsp > 0

            changed = jnp.any(segsp != prev)

            def do_flush(prev, aS, aM):
                flush(prev, aS, aM)
                return ([jnp.zeros((L,), jnp.float32)] * NK,
                        [jnp.full((L,), NEG_INF, jnp.float32)] * NK)

            aS, aM = lax.cond(
                changed, do_flush, lambda p, s, m: (list(s), list(m)),
                prev, tuple(aS), tuple(aM))
            aS = [aS[k] + ar[k] * score for k in range(NK)]
            aM = [jnp.maximum(aM[k], jnp.where(mask, ar[k], NEG_INF))
                  for k in range(NK)]
            return (segsp, *aS, *aM)

        def process(blk, p, carry):
            cur_i, nxt_i = idx_bufs[p], idx_bufs[1 - p]
            cur_r, nxt_r = rows_bufs[p], rows_bufs[1 - p]
            pltpu.make_async_copy(
                idxc_hbm.at[pl.ds(0, IW)], nxt_i, semi[1 - p]).wait()
            pltpu.async_copy(
                hh_hbm.at[nxt_i.at[pl.ds(0, 2 * B)]], nxt_r, semr[1 - p])
            segs, validfs = [], []
            gbase = wid * PW + blk * B
            for g in range(NG):
                src16 = cur_i[pl.ds(g * L, L)]
                segs.append(plsc.load_gather(batch_v, [src16]))
                vi = cur_i[pl.ds(2 * B + g * L, L)]
                gi = iota + (gbase + g * L)
                is_self = (gi >= E) & (gi < E + N)
                pres16 = plsc.load_gather(pres_v, [src16])
                selfv = jnp.where(pres16 == 0, 1, 0)
                validfs.append(
                    jnp.where(is_self, selfv, vi).astype(jnp.float32))
            pltpu.async_copy(
                idxc_hbm.at[pl.ds((base0 + blk + 2) * IW, IW)],
                cur_i, semi[p])
            pltpu.make_async_copy(
                hh_hbm.at[pl.ds(0, 2 * B)], cur_r, semr[p]).wait()

            def lane_body(ep, c):
                for g in range(NG):
                    c = edge_work(cur_r, g, ep, segs, validfs, c)
                return c

            return lax.fori_loop(0, L, lane_body, carry)

        def block_body(i, carry):
            for p in range(2):
                carry = process(2 * i + p, p, carry)
            return carry

        carry0 = (jnp.zeros((L,), jnp.int32),
                  *([zeros16] * NK), *([neginf16] * NK))
        carry = lax.fori_loop(0, NBW // 2, block_body, carry0)
        flush(carry[0], carry[1:1 + NK], carry[1 + NK:])

        pltpu.make_async_copy(hh_hbm.at[pl.ds(0, 2 * B)], rows0, semr0).wait()
        pltpu.make_async_copy(idxc_hbm.at[pl.ds(0, IW)], idx1_v, semi1).wait()

        pltpu.sync_copy(accs, out_s_hbm.at[wid])
        pltpu.sync_copy(accm, out_m_hbm.at[wid])

    return edge_kernel


def kernel(x, edge_index, edge_attr, pos, batch, W, b, attn, Wr, br):
    num_nodes = x.shape[0]
    E = edge_index.shape[1]

    a = jnp.minimum(edge_index[0], edge_index[1])
    bb = jnp.maximum(edge_index[0], edge_index[1])
    ids = a * num_nodes + bb
    ids_sorted = jnp.sort(ids)
    keep = jnp.concatenate(
        [jnp.ones((1,), dtype=bool), ids_sorted[1:] != ids_sorted[:-1]])
    a_s = (ids_sorted // num_nodes).astype(jnp.int32)
    b_s = (ids_sorted % num_nodes).astype(jnp.int32)
    all_nodes = jnp.arange(num_nodes, dtype=jnp.int32)
    ET = E + num_nodes
    NBW = -(-ET // (NW * B))
    NBW += NBW % 2
    PW = NBW * B
    EP = NW * PW
    pad = EP - ET
    src_pad = jnp.concatenate([a_s, all_nodes, jnp.zeros((pad,), jnp.int32)])
    dst_pad = jnp.concatenate([b_s, all_nodes, jnp.zeros((pad,), jnp.int32)])
    val_pad = jnp.concatenate(
        [keep, jnp.ones((num_nodes,), bool),
         jnp.zeros((pad,), bool)]).astype(jnp.int32)
    NBT = NW * NBW
    idxc = jnp.concatenate(
        [src_pad.reshape(NBT, B), dst_pad.reshape(NBT, B),
         val_pad.reshape(NBT, B)], axis=1)
    idxc = jnp.concatenate(
        [idxc.reshape(-1), jnp.zeros((2 * IW,), jnp.int32)])

    NPAD = -(-num_nodes // 64) * 64
    batch_pad = jnp.concatenate(
        [batch.astype(jnp.int32),
         jnp.zeros((NPAD - num_nodes,), jnp.int32)])
    params = jnp.concatenate(
        [attn[0].astype(jnp.float32), Wr[0].astype(jnp.float32),
         br.astype(jnp.float32), jnp.zeros((272 - 2 * D - 1,), jnp.float32)])

    hh = _make_table(x, W, b)
    endpoints = edge_index.reshape(-1).astype(jnp.int32)
    edge_kernel = _make_edge_kernel(NBW, NPAD, E, num_nodes)
    out_s, out_m = edge_kernel(hh, idxc, batch_pad, params, endpoints)

    out1 = out_s.sum(axis=0).reshape(G, D)
    out2 = out_m.max(axis=0).reshape(G, D)
    return jnp.concatenate([out1, out2], axis=1)
